# baseline (speedup 1.0000x reference)
import jax
import jax.numpy as jnp
from jax import lax
from jax.experimental import pallas as pl
from jax.experimental.pallas import tpu as pltpu
from jax.experimental.pallas import tpu_sc as plsc

F = 128
F2 = 64
LANES = 16
NC = 2
NS = 16
K = 128


def _ceil_to(x, m):
    return ((x + m - 1) // m) * m


def _make_spmm(e_pad, n_pad, mode, n=None):
    nchunk = e_pad // K // NS
    rpt = n_pad // NS
    mesh = plsc.VectorSubcoreMesh(core_axis_name="c", subcore_axis_name="s")

    SB = 40
    D = 8
    H = D // 2

    def body(rows_hbm, cols_hbm, vals_hbm, x_hbm, zeros_hbm, *rest):
        if mode == "diag":
            diag_hbm, out_hbm = rest[0], rest[1]
            rest = rest[2:]
        else:
            out_hbm = rest[0]
            rest = rest[1:]
        colw, roww, valw = rest[0], rest[1], rest[2]
        rest = rest[3:]
        bufs = list(rest[:D])
        acc, gsem, ssem = rest[D], rest[D + 1], rest[D + 2]
        if mode == "diag":
            dbuf, dv = rest[D + 3], rest[D + 4]
        cid = lax.axis_index("c")
        sid = lax.axis_index("s")
        r0 = pl.multiple_of(sid * rpt, 8)
        pltpu.sync_copy(zeros_hbm.at[pl.ds(r0, rpt)], acc.at[pl.ds(r0, rpt)])
        c0 = pl.multiple_of(sid * nchunk, 8)
        plsc.subcore_barrier()

        x_c = x_hbm.at[cid]

        def gwait(buf):
            pltpu.make_async_copy(zeros_hbm.at[pl.ds(0, K)], buf, gsem).wait()

        def swait(buf):
            pltpu.make_async_copy(zeros_hbm.at[pl.ds(0, K)], buf, ssem).wait()

        nj = F2 // LANES

        def scale(buf, t):
            def group(g, _):
                vv = valw[t, pl.ds(g * LANES, LANES)]
                for l0 in range(0, LANES, 4):
                    vs = [vv[l0 + i] for i in range(4)]
                    xs = [buf[g * LANES + l0 + i, pl.ds(j * LANES, LANES)]
                          for i in range(4) for j in range(nj)]
                    ys = [xs[i * nj + j] * vs[i]
                          for i in range(4) for j in range(nj)]
                    for i in range(4):
                        for j in range(nj):
                            buf[g * LANES + l0 + i, pl.ds(j * LANES, LANES)] \
                                = ys[i * nj + j]
                return 0

            lax.fori_loop(0, K // LANES, group, 0)

        nD = SB // D

        def superblock(sb, _):
            cb = pl.multiple_of(c0 + sb * SB, 8)
            pltpu.sync_copy(rows_hbm.at[pl.ds(cb, SB)], roww)
            pltpu.sync_copy(cols_hbm.at[pl.ds(cb, SB)], colw)
            pltpu.sync_copy(vals_hbm.at[pl.ds(cb, SB)], valw)
            for i in range(H):
                pltpu.async_copy(x_c.at[colw.at[i]], bufs[i], gsem)

            def step(tD, _):
                for i in range(D):
                    t = tD * D + i
                    b = bufs[i]
                    gwait(b)
                    scale(b, t)
                    pltpu.async_copy(b, acc.at[roww.at[t]], ssem, add=True)
                    if i < H:
                        @pl.when(tD > 0)
                        def _():
                            swait(b)
                    else:
                        swait(b)
                    if i < H:
                        pltpu.async_copy(x_c.at[colw.at[t + H]], bufs[i + H],
                                         gsem)
                    else:
                        @pl.when(tD < nD - 1)
                        def _():
                            pltpu.async_copy(x_c.at[colw.at[t + H]],
                                             bufs[i - H], gsem)
                return 0

            lax.fori_loop(0, nD, step, 0)
            for i in range(H):
                swait(bufs[i])
            return 0

        lax.fori_loop(0, nchunk // SB, superblock, 0)
        plsc.subcore_barrier()

        if mode == "plain":
            pltpu.sync_copy(acc.at[pl.ds(r0, rpt)],
                            out_hbm.at[cid, pl.ds(r0, rpt)])
        elif mode == "concat":
            last = n - (NS - 1) * rpt
            csel = pl.ds(cid * F2, F2)

            @pl.when(sid < NS - 1)
            def _():
                pltpu.sync_copy(acc.at[pl.ds(r0, rpt)],
                                out_hbm.at[pl.ds(r0, rpt), csel])

            @pl.when(sid == NS - 1)
            def _():
                pltpu.sync_copy(acc.at[pl.ds(r0, last)],
                                out_hbm.at[pl.ds(r0, last), csel])
        else:
            def dump_blk(blk, _):
                rb = pl.multiple_of(r0 + blk * 64, 8)
                pltpu.sync_copy(acc.at[pl.ds(rb, 64)], dbuf)
                pltpu.sync_copy(diag_hbm.at[pl.ds(rb, 64)], dv)
                for g in range(4):
                    dvv = dv[pl.ds(g * LANES, LANES)]
                    for l0 in range(0, LANES, 4):
                        vs = [dvv[l0 + i] for i in range(4)]
                        xs = [dbuf[g * LANES + l0 + i, pl.ds(j * LANES, LANES)]
                              for i in range(4) for j in range(nj)]
                        ys = [xs[i * nj + j] * vs[i]
                              for i in range(4) for j in range(nj)]
                        for i in range(4):
                            for j in range(nj):
                                dbuf[g * LANES + l0 + i,
                                     pl.ds(j * LANES, LANES)] = ys[i * nj + j]
                pltpu.sync_copy(dbuf, out_hbm.at[cid, pl.ds(rb, 64)])
                return 0

            lax.fori_loop(0, rpt // 64, dump_blk, 0)

    if mode == "concat":
        out_type = jax.ShapeDtypeStruct((n, F), jnp.float32)
    else:
        out_type = jax.ShapeDtypeStruct((NC, n_pad, F2), jnp.float32)

    return pl.kernel(
        body,
        out_type=out_type,
        mesh=mesh,
        compiler_params=pltpu.CompilerParams(use_tc_tiling_on_sc=False),
        scratch_types=[
            pltpu.VMEM((40, K), jnp.int32),
            pltpu.VMEM((40, K), jnp.int32),
            pltpu.VMEM((40, K), jnp.float32),
            *[pltpu.VMEM((K, F2), jnp.float32) for _ in range(8)],
            pltpu.VMEM_SHARED((n_pad, F2), jnp.float32),
            pltpu.SemaphoreType.DMA,
            pltpu.SemaphoreType.DMA,
            *([pltpu.VMEM((64, F2), jnp.float32),
               pltpu.VMEM((64,), jnp.float32)]
              if mode == "diag" else []),
        ],
    )


def _make_densify(e_pad, nf_pad):
    nch_t = e_pad // K // NS
    nsup = nch_t // 8
    rpt = nf_pad // NS
    mesh = plsc.VectorSubcoreMesh(core_axis_name="c", subcore_axis_name="s",
                                  num_cores=1)

    def body(rows_hbm, cols_hbm, vals_hbm, zeros_hbm, out_hbm,
             rw, cw, valw, idxw, acc, dsem):
        sid = lax.axis_index("s")
        r0 = pl.multiple_of(sid * rpt, 8)
        pltpu.sync_copy(zeros_hbm.at[pl.ds(r0, rpt)], acc.at[pl.ds(r0, rpt)])
        plsc.subcore_barrier()

        def sup(s8, _):
            base = pl.multiple_of(sid * nch_t + s8 * 8, 8)
            pltpu.sync_copy(rows_hbm.at[pl.ds(base, 8)], rw)
            pltpu.sync_copy(cols_hbm.at[pl.ds(base, 8)], cw)
            pltpu.sync_copy(vals_hbm.at[pl.ds(base, 8)], valw)
            for j in range(8):
                for g in range(K // LANES):
                    sl = pl.ds(g * LANES, LANES)
                    idxw[j, sl] = rw[j, sl] * F + cw[j, sl]
                pltpu.async_copy(valw.at[j], acc.at[idxw.at[j]], dsem,
                                 add=True)
            for j in range(8):
                pltpu.make_async_copy(zeros_hbm.at[pl.ds(0, K)], valw.at[j],
                                      dsem).wait()
            return 0

        lax.fori_loop(0, nsup, sup, 0)
        plsc.subcore_barrier()
        pltpu.sync_copy(acc.at[pl.ds(r0, rpt)], out_hbm.at[pl.ds(r0, rpt)])

    return pl.kernel(
        body,
        out_type=jax.ShapeDtypeStruct((nf_pad,), jnp.float32),
        mesh=mesh,
        compiler_params=pltpu.CompilerParams(use_tc_tiling_on_sc=False),
        scratch_types=[
            pltpu.VMEM((8, K), jnp.int32),
            pltpu.VMEM((8, K), jnp.int32),
            pltpu.VMEM((8, K), jnp.float32),
            pltpu.VMEM((8, K), jnp.int32),
            pltpu.VMEM_SHARED((nf_pad,), jnp.float32),
            pltpu.SemaphoreType.DMA,
        ],
    )


_BM = 1024


def _tc_mm1(z, w):
    np_ = z.shape[0]

    def body(z_ref, w_ref, o_ref):
        y = jnp.dot(z_ref[...], w_ref[...], preferred_element_type=jnp.float32)
        o_ref[0] = y[:, :F2]
        o_ref[1] = y[:, F2:]

    return pl.pallas_call(
        body,
        grid=(np_ // _BM,),
        in_specs=[
            pl.BlockSpec((_BM, F), lambda i: (i, 0)),
            pl.BlockSpec((F, F), lambda i: (0, 0)),
        ],
        out_specs=pl.BlockSpec((NC, _BM, F2), lambda i: (0, i, 0)),
        out_shape=jax.ShapeDtypeStruct((NC, np_, F2), jnp.float32),
    )(z, w)


def _tc_mm2(p, w):
    np_ = p.shape[1]

    def body(p_ref, w_ref, o_ref):
        x = jnp.concatenate([p_ref[0], p_ref[1]], axis=-1)
        x = jnp.maximum(x, 0.0)
        y = jnp.dot(x, w_ref[...], preferred_element_type=jnp.float32)
        o_ref[0] = y[:, :F2]
        o_ref[1] = y[:, F2:]

    return pl.pallas_call(
        body,
        grid=(np_ // _BM,),
        in_specs=[
            pl.BlockSpec((NC, _BM, F2), lambda i: (0, i, 0)),
            pl.BlockSpec((F, F), lambda i: (0, 0)),
        ],
        out_specs=pl.BlockSpec((NC, _BM, F2), lambda i: (0, i, 0)),
        out_shape=jax.ShapeDtypeStruct((NC, np_, F2), jnp.float32),
    )(p, w)


def _tc_scale(p, d):
    np_ = p.shape[1]

    def body(p_ref, d_ref, o_ref):
        o_ref[...] = p_ref[...] * d_ref[...][None]

    return pl.pallas_call(
        body,
        grid=(np_ // _BM,),
        in_specs=[
            pl.BlockSpec((NC, _BM, F2), lambda i: (0, i, 0)),
            pl.BlockSpec((_BM, 1), lambda i: (i, 0)),
        ],
        out_specs=pl.BlockSpec((NC, _BM, F2), lambda i: (0, i, 0)),
        out_shape=jax.ShapeDtypeStruct((NC, np_, F2), jnp.float32),
    )(p, d)


def kernel(phi_indices, phi_values, phi_inverse_indices, phi_inverse_values,
           feature_indices, feature_values, W1, diag_w1, W2, diag_w2):
    n = diag_w1.shape[0]
    n_pad = _ceil_to(n, 512)
    e_pad = _ceil_to(phi_values.shape[0], NS * K * 8)
    ef_pad = _ceil_to(feature_values.shape[0], NS * K * 8)
    nf_pad = n_pad * F

    def pad_chunks(x, tot):
        return jnp.pad(x, (0, tot - x.shape[0])).reshape(tot // K, K)

    pr = pad_chunks(phi_indices[0], e_pad)
    pc = pad_chunks(phi_indices[1], e_pad)
    pv = pad_chunks(phi_values, e_pad)
    qr = pad_chunks(phi_inverse_indices[0], e_pad)
    qc = pad_chunks(phi_inverse_indices[1], e_pad)
    qv = pad_chunks(phi_inverse_values, e_pad)
    fr = pad_chunks(feature_indices[0], ef_pad)
    fc = pad_chunks(feature_indices[1], ef_pad)
    fv = pad_chunks(feature_values, ef_pad)

    zeros2d = jnp.zeros((n_pad, F2), jnp.float32)
    zeros1d = jnp.zeros((nf_pad,), jnp.float32)
    d1 = jnp.pad(diag_w1, (0, n_pad - n))[:, None]
    d2 = jnp.pad(diag_w2, (0, n_pad - n))[:, None]

    spmm_plain = _make_spmm(e_pad, n_pad, "plain")
    spmm_cat = _make_spmm(e_pad, n_pad, "concat", n=n)
    densify = _make_densify(ef_pad, nf_pad)

    z = densify(fr, fc, fv, zeros1d).reshape(n_pad, F)
    f1 = _tc_mm1(z, W1)
    p = spmm_plain(qr, qc, qv, f1, zeros2d)
    t1 = _tc_scale(p, d1)
    p = spmm_plain(pr, pc, pv, t1, zeros2d)
    f2 = _tc_mm2(p, W2)
    p = spmm_plain(qr, qc, qv, f2, zeros2d)
    t2 = _tc_scale(p, d2)
    return spmm_cat(pr, pc, pv, t2, zeros2d)


# speedup vs baseline: 1.1318x; 1.1318x over previous
import jax
import jax.numpy as jnp
from jax import lax
from jax.experimental import pallas as pl
from jax.experimental.pallas import tpu as pltpu
from jax.experimental.pallas import tpu_sc as plsc

F = 128
F2 = 64
LANES = 16
NC = 2
NS = 16
K = 128


def _ceil_to(x, m):
    return ((x + m - 1) // m) * m


def _make_spmm(e_pad, n_pad, mode, n=None):
    nchunk = e_pad // K // NS
    rpt = n_pad // NS
    mesh = plsc.VectorSubcoreMesh(core_axis_name="c", subcore_axis_name="s")

    SB = 40
    D = 8
    H = D // 2

    def body(rows_hbm, cols_hbm, vals_hbm, x_hbm, zeros_hbm, *rest):
        if mode == "diag":
            diag_hbm, out_hbm = rest[0], rest[1]
            rest = rest[2:]
        else:
            out_hbm = rest[0]
            rest = rest[1:]
        colw, roww, valw = rest[0], rest[1], rest[2]
        rest = rest[3:]
        bufs = list(rest[:D])
        acc, gsem, ssem = rest[D], rest[D + 1], rest[D + 2]
        if mode == "diag":
            dbuf, dv = rest[D + 3], rest[D + 4]
        cid = lax.axis_index("c")
        sid = lax.axis_index("s")
        r0 = pl.multiple_of(sid * rpt, 8)
        pltpu.sync_copy(zeros_hbm.at[pl.ds(r0, rpt)], acc.at[pl.ds(r0, rpt)])
        c0 = pl.multiple_of(sid * nchunk, 8)
        plsc.subcore_barrier()

        x_c = x_hbm.at[cid]

        def gwait(buf):
            pltpu.make_async_copy(zeros_hbm.at[pl.ds(0, K)], buf, gsem).wait()

        def swait(buf):
            pltpu.make_async_copy(zeros_hbm.at[pl.ds(0, K)], buf, ssem).wait()

        nj = F2 // LANES

        def scale(buf, t):
            def group(g, _):
                vv = valw[t, pl.ds(g * LANES, LANES)]
                for l0 in range(0, LANES, 4):
                    vs = [vv[l0 + i] for i in range(4)]
                    xs = [buf[g * LANES + l0 + i, pl.ds(j * LANES, LANES)]
                          for i in range(4) for j in range(nj)]
                    ys = [xs[i * nj + j] * vs[i]
                          for i in range(4) for j in range(nj)]
                    for i in range(4):
                        for j in range(nj):
                            buf[g * LANES + l0 + i, pl.ds(j * LANES, LANES)] \
                                = ys[i * nj + j]
                return 0

            lax.fori_loop(0, K // LANES, group, 0)

        nD = SB // D

        def superblock(sb, _):
            cb = pl.multiple_of(c0 + sb * SB, 8)
            pltpu.sync_copy(rows_hbm.at[pl.ds(cb, SB)], roww)
            pltpu.sync_copy(cols_hbm.at[pl.ds(cb, SB)], colw)
            pltpu.sync_copy(vals_hbm.at[pl.ds(cb, SB)], valw)
            for i in range(H):
                pltpu.async_copy(x_c.at[colw.at[i]], bufs[i], gsem)

            def step(tD, _):
                for i in range(D):
                    t = tD * D + i
                    b = bufs[i]
                    gwait(b)
                    scale(b, t)
                    pltpu.async_copy(b, acc.at[roww.at[t]], ssem, add=True)
                    if i < H:
                        @pl.when(tD > 0)
                        def _():
                            swait(b)
                    else:
                        swait(b)
                    if i < H:
                        pltpu.async_copy(x_c.at[colw.at[t + H]], bufs[i + H],
                                         gsem)
                    else:
                        @pl.when(tD < nD - 1)
                        def _():
                            pltpu.async_copy(x_c.at[colw.at[t + H]],
                                             bufs[i - H], gsem)
                return 0

            lax.fori_loop(0, nD, step, 0)
            for i in range(H):
                swait(bufs[i])
            return 0

        lax.fori_loop(0, nchunk // SB, superblock, 0)
        plsc.subcore_barrier()

        if mode == "plain":
            pltpu.sync_copy(acc.at[pl.ds(r0, rpt)],
                            out_hbm.at[cid, pl.ds(r0, rpt)])
        elif mode == "concat":
            last = n - (NS - 1) * rpt
            csel = pl.ds(cid * F2, F2)

            @pl.when(sid < NS - 1)
            def _():
                pltpu.sync_copy(acc.at[pl.ds(r0, rpt)],
                                out_hbm.at[pl.ds(r0, rpt), csel])

            @pl.when(sid == NS - 1)
            def _():
                pltpu.sync_copy(acc.at[pl.ds(r0, last)],
                                out_hbm.at[pl.ds(r0, last), csel])
        else:
            def dump_blk(blk, _):
                rb = pl.multiple_of(r0 + blk * 64, 8)
                pltpu.sync_copy(acc.at[pl.ds(rb, 64)], dbuf)
                pltpu.sync_copy(diag_hbm.at[pl.ds(rb, 64)], dv)
                for g in range(4):
                    dvv = dv[pl.ds(g * LANES, LANES)]
                    for l0 in range(0, LANES, 4):
                        vs = [dvv[l0 + i] for i in range(4)]
                        xs = [dbuf[g * LANES + l0 + i, pl.ds(j * LANES, LANES)]
                              for i in range(4) for j in range(nj)]
                        ys = [xs[i * nj + j] * vs[i]
                              for i in range(4) for j in range(nj)]
                        for i in range(4):
                            for j in range(nj):
                                dbuf[g * LANES + l0 + i,
                                     pl.ds(j * LANES, LANES)] = ys[i * nj + j]
                pltpu.sync_copy(dbuf, out_hbm.at[cid, pl.ds(rb, 64)])
                return 0

            lax.fori_loop(0, rpt // 64, dump_blk, 0)

    if mode == "concat":
        out_type = jax.ShapeDtypeStruct((n, F), jnp.float32)
    else:
        out_type = jax.ShapeDtypeStruct((NC, n_pad, F2), jnp.float32)

    return pl.kernel(
        body,
        out_type=out_type,
        mesh=mesh,
        compiler_params=pltpu.CompilerParams(use_tc_tiling_on_sc=False),
        scratch_types=[
            pltpu.VMEM((40, K), jnp.int32),
            pltpu.VMEM((40, K), jnp.int32),
            pltpu.VMEM((40, K), jnp.float32),
            *[pltpu.VMEM((K, F2), jnp.float32) for _ in range(8)],
            pltpu.VMEM_SHARED((n_pad, F2), jnp.float32),
            pltpu.SemaphoreType.DMA,
            pltpu.SemaphoreType.DMA,
            *([pltpu.VMEM((64, F2), jnp.float32),
               pltpu.VMEM((64,), jnp.float32)]
              if mode == "diag" else []),
        ],
    )


def _make_densify(e_pad, nf_pad):
    nch_t = e_pad // K // NS
    nsup = nch_t // 8
    rpt = nf_pad // NS
    mesh = plsc.VectorSubcoreMesh(core_axis_name="c", subcore_axis_name="s",
                                  num_cores=1)

    def body(rows_hbm, cols_hbm, vals_hbm, zeros_hbm, out_hbm,
             rw, cw, valw, idxw, acc, dsem):
        sid = lax.axis_index("s")
        r0 = pl.multiple_of(sid * rpt, 8)
        pltpu.sync_copy(zeros_hbm.at[pl.ds(r0, rpt)], acc.at[pl.ds(r0, rpt)])
        plsc.subcore_barrier()

        def sup(s8, _):
            base = pl.multiple_of(sid * nch_t + s8 * 8, 8)
            pltpu.sync_copy(rows_hbm.at[pl.ds(base, 8)], rw)
            pltpu.sync_copy(cols_hbm.at[pl.ds(base, 8)], cw)
            pltpu.sync_copy(vals_hbm.at[pl.ds(base, 8)], valw)
            for j in range(8):
                for g in range(K // LANES):
                    sl = pl.ds(g * LANES, LANES)
                    idxw[j, sl] = rw[j, sl] * F + cw[j, sl]
                pltpu.async_copy(valw.at[j], acc.at[idxw.at[j]], dsem,
                                 add=True)
            for j in range(8):
                pltpu.make_async_copy(zeros_hbm.at[pl.ds(0, K)], valw.at[j],
                                      dsem).wait()
            return 0

        lax.fori_loop(0, nsup, sup, 0)
        plsc.subcore_barrier()
        pltpu.sync_copy(acc.at[pl.ds(r0, rpt)], out_hbm.at[pl.ds(r0, rpt)])

    return pl.kernel(
        body,
        out_type=jax.ShapeDtypeStruct((nf_pad,), jnp.float32),
        mesh=mesh,
        compiler_params=pltpu.CompilerParams(use_tc_tiling_on_sc=False),
        scratch_types=[
            pltpu.VMEM((8, K), jnp.int32),
            pltpu.VMEM((8, K), jnp.int32),
            pltpu.VMEM((8, K), jnp.float32),
            pltpu.VMEM((8, K), jnp.int32),
            pltpu.VMEM_SHARED((nf_pad,), jnp.float32),
            pltpu.SemaphoreType.DMA,
        ],
    )


_BM = 1024


def _tc_mm1(z, w):
    np_ = z.shape[0]

    def body(z_ref, w_ref, o_ref):
        y = jnp.dot(z_ref[...], w_ref[...], preferred_element_type=jnp.float32)
        o_ref[0] = y[:, :F2]
        o_ref[1] = y[:, F2:]

    return pl.pallas_call(
        body,
        grid=(np_ // _BM,),
        in_specs=[
            pl.BlockSpec((_BM, F), lambda i: (i, 0)),
            pl.BlockSpec((F, F), lambda i: (0, 0)),
        ],
        out_specs=pl.BlockSpec((NC, _BM, F2), lambda i: (0, i, 0)),
        out_shape=jax.ShapeDtypeStruct((NC, np_, F2), jnp.float32),
    )(z, w)


def _tc_mm2(p, w):
    np_ = p.shape[1]

    def body(p_ref, w_ref, o_ref):
        x = jnp.concatenate([p_ref[0], p_ref[1]], axis=-1)
        x = jnp.maximum(x, 0.0)
        y = jnp.dot(x, w_ref[...], preferred_element_type=jnp.float32)
        o_ref[0] = y[:, :F2]
        o_ref[1] = y[:, F2:]

    return pl.pallas_call(
        body,
        grid=(np_ // _BM,),
        in_specs=[
            pl.BlockSpec((NC, _BM, F2), lambda i: (0, i, 0)),
            pl.BlockSpec((F, F), lambda i: (0, 0)),
        ],
        out_specs=pl.BlockSpec((NC, _BM, F2), lambda i: (0, i, 0)),
        out_shape=jax.ShapeDtypeStruct((NC, np_, F2), jnp.float32),
    )(p, w)


def _tc_scale(p, d):
    np_ = p.shape[1]

    def body(p_ref, d_ref, o_ref):
        o_ref[...] = p_ref[...] * d_ref[...][None]

    return pl.pallas_call(
        body,
        grid=(np_ // _BM,),
        in_specs=[
            pl.BlockSpec((NC, _BM, F2), lambda i: (0, i, 0)),
            pl.BlockSpec((_BM, 1), lambda i: (i, 0)),
        ],
        out_specs=pl.BlockSpec((NC, _BM, F2), lambda i: (0, i, 0)),
        out_shape=jax.ShapeDtypeStruct((NC, np_, F2), jnp.float32),
    )(p, d)


def _tc_final(p, n):
    bm = 2000

    def body(p_ref, o_ref):
        o_ref[...] = jnp.concatenate([p_ref[0], p_ref[1]], axis=-1)

    return pl.pallas_call(
        body,
        grid=(n // bm,),
        in_specs=[pl.BlockSpec((NC, bm, F2), lambda i: (0, i, 0))],
        out_specs=pl.BlockSpec((bm, F), lambda i: (i, 0)),
        out_shape=jax.ShapeDtypeStruct((n, F), jnp.float32),
    )(p)


def kernel(phi_indices, phi_values, phi_inverse_indices, phi_inverse_values,
           feature_indices, feature_values, W1, diag_w1, W2, diag_w2):
    n = diag_w1.shape[0]
    n_pad = _ceil_to(n, 512)
    e_pad = _ceil_to(phi_values.shape[0], NS * K * 8)
    ef_pad = _ceil_to(feature_values.shape[0], NS * K * 8)
    nf_pad = n_pad * F

    def pad_chunks(x, tot):
        return jnp.pad(x, (0, tot - x.shape[0])).reshape(tot // K, K)

    pr = pad_chunks(phi_indices[0], e_pad)
    pc = pad_chunks(phi_indices[1], e_pad)
    pv = pad_chunks(phi_values, e_pad)
    qr = pad_chunks(phi_inverse_indices[0], e_pad)
    qc = pad_chunks(phi_inverse_indices[1], e_pad)
    qv = pad_chunks(phi_inverse_values, e_pad)
    fr = pad_chunks(feature_indices[0], ef_pad)
    fc = pad_chunks(feature_indices[1], ef_pad)
    fv = pad_chunks(feature_values, ef_pad)

    zeros2d = jnp.zeros((n_pad, F2), jnp.float32)
    zeros1d = jnp.zeros((nf_pad,), jnp.float32)
    d1 = jnp.pad(diag_w1, (0, n_pad - n))
    d2 = jnp.pad(diag_w2, (0, n_pad - n))

    spmm_diag = _make_spmm(e_pad, n_pad, "diag")
    spmm_plain = _make_spmm(e_pad, n_pad, "plain")
    densify = _make_densify(ef_pad, nf_pad)

    z = densify(fr, fc, fv, zeros1d).reshape(n_pad, F)
    f1 = _tc_mm1(z, W1)
    p = spmm_diag(qr, qc, qv, f1, zeros2d, d1)
    p = spmm_plain(pr, pc, pv, p, zeros2d)
    f2 = _tc_mm2(p, W2)
    p = spmm_diag(qr, qc, qv, f2, zeros2d, d2)
    p = spmm_plain(pr, pc, pv, p, zeros2d)
    return _tc_final(p, n)


# speedup vs baseline: 1.1669x; 1.0310x over previous
import jax
import jax.numpy as jnp
from jax import lax
from jax.experimental import pallas as pl
from jax.experimental.pallas import tpu as pltpu
from jax.experimental.pallas import tpu_sc as plsc

F = 128
F2 = 64
LANES = 16
NC = 2
NS = 16
K = 128


def _ceil_to(x, m):
    return ((x + m - 1) // m) * m


def _make_spmm(e_pad, n_pad, mode, n=None):
    nchunk = e_pad // K // NS
    rpt = n_pad // NS
    mesh = plsc.VectorSubcoreMesh(core_axis_name="c", subcore_axis_name="s")

    SB = 40
    D = 8
    H = D // 2

    def body(rows_hbm, cols_hbm, vals_hbm, x_hbm, zeros_hbm, *rest):
        if mode == "diag":
            diag_hbm, out_hbm = rest[0], rest[1]
            rest = rest[2:]
        else:
            out_hbm = rest[0]
            rest = rest[1:]
        colw, roww, valw = rest[0], rest[1], rest[2]
        rest = rest[3:]
        bufs = list(rest[:D])
        acc, gsem, ssem = rest[D], rest[D + 1], rest[D + 2]
        if mode == "diag":
            dbuf, dv = rest[D + 3], rest[D + 4]
        cid = lax.axis_index("c")
        sid = lax.axis_index("s")
        r0 = pl.multiple_of(sid * rpt, 8)
        pltpu.sync_copy(zeros_hbm.at[pl.ds(r0, rpt)], acc.at[pl.ds(r0, rpt)])
        c0 = pl.multiple_of(sid * nchunk, 8)
        plsc.subcore_barrier()

        x_c = x_hbm.at[cid]

        def gwait(buf):
            pltpu.make_async_copy(zeros_hbm.at[pl.ds(0, K)], buf, gsem).wait()

        def swait(buf):
            pltpu.make_async_copy(zeros_hbm.at[pl.ds(0, K)], buf, ssem).wait()

        nj = F2 // LANES

        def scale(buf, t):
            def group(g, _):
                vv = valw[t, pl.ds(g * LANES, LANES)]
                for l0 in range(0, LANES, 4):
                    vs = [vv[l0 + i] for i in range(4)]
                    xs = [buf[g * LANES + l0 + i, pl.ds(j * LANES, LANES)]
                          for i in range(4) for j in range(nj)]
                    ys = [xs[i * nj + j] * vs[i]
                          for i in range(4) for j in range(nj)]
                    for i in range(4):
                        for j in range(nj):
                            buf[g * LANES + l0 + i, pl.ds(j * LANES, LANES)] \
                                = ys[i * nj + j]
                return 0

            lax.fori_loop(0, K // LANES, group, 0)

        nD = SB // D

        def superblock(sb, _):
            cb = pl.multiple_of(c0 + sb * SB, 8)
            pltpu.sync_copy(rows_hbm.at[pl.ds(cb, SB)], roww)
            pltpu.sync_copy(cols_hbm.at[pl.ds(cb, SB)], colw)
            pltpu.sync_copy(vals_hbm.at[pl.ds(cb, SB)], valw)
            for i in range(H):
                pltpu.async_copy(x_c.at[colw.at[i]], bufs[i], gsem)

            def step(tD, _):
                for i in range(D):
                    t = tD * D + i
                    b = bufs[i]
                    gwait(b)
                    scale(b, t)
                    pltpu.async_copy(b, acc.at[roww.at[t]], ssem, add=True)
                    if i < H:
                        @pl.when(tD > 0)
                        def _():
                            swait(b)
                    else:
                        swait(b)
                    if i < H:
                        pltpu.async_copy(x_c.at[colw.at[t + H]], bufs[i + H],
                                         gsem)
                    else:
                        @pl.when(tD < nD - 1)
                        def _():
                            pltpu.async_copy(x_c.at[colw.at[t + H]],
                                             bufs[i - H], gsem)
                return 0

            lax.fori_loop(0, nD, step, 0)
            for i in range(H):
                swait(bufs[i])
            return 0

        lax.fori_loop(0, nchunk // SB, superblock, 0)
        plsc.subcore_barrier()

        if mode == "plain":
            pltpu.sync_copy(acc.at[pl.ds(r0, rpt)],
                            out_hbm.at[cid, pl.ds(r0, rpt)])
        elif mode == "concat":
            last = n - (NS - 1) * rpt
            csel = pl.ds(cid * F2, F2)

            @pl.when(sid < NS - 1)
            def _():
                pltpu.sync_copy(acc.at[pl.ds(r0, rpt)],
                                out_hbm.at[pl.ds(r0, rpt), csel])

            @pl.when(sid == NS - 1)
            def _():
                pltpu.sync_copy(acc.at[pl.ds(r0, last)],
                                out_hbm.at[pl.ds(r0, last), csel])
        else:
            def dump_blk(blk, _):
                rb = pl.multiple_of(r0 + blk * 64, 8)
                pltpu.sync_copy(acc.at[pl.ds(rb, 64)], dbuf)
                pltpu.sync_copy(diag_hbm.at[pl.ds(rb, 64)], dv)
                for g in range(4):
                    dvv = dv[pl.ds(g * LANES, LANES)]
                    for l0 in range(0, LANES, 4):
                        vs = [dvv[l0 + i] for i in range(4)]
                        xs = [dbuf[g * LANES + l0 + i, pl.ds(j * LANES, LANES)]
                              for i in range(4) for j in range(nj)]
                        ys = [xs[i * nj + j] * vs[i]
                              for i in range(4) for j in range(nj)]
                        for i in range(4):
                            for j in range(nj):
                                dbuf[g * LANES + l0 + i,
                                     pl.ds(j * LANES, LANES)] = ys[i * nj + j]
                pltpu.sync_copy(dbuf, out_hbm.at[cid, pl.ds(rb, 64)])
                return 0

            lax.fori_loop(0, rpt // 64, dump_blk, 0)

    if mode == "concat":
        out_type = jax.ShapeDtypeStruct((n, F), jnp.float32)
    else:
        out_type = jax.ShapeDtypeStruct((NC, n_pad, F2), jnp.float32)

    return pl.kernel(
        body,
        out_type=out_type,
        mesh=mesh,
        compiler_params=pltpu.CompilerParams(use_tc_tiling_on_sc=False),
        scratch_types=[
            pltpu.VMEM((40, K), jnp.int32),
            pltpu.VMEM((40, K), jnp.int32),
            pltpu.VMEM((40, K), jnp.float32),
            *[pltpu.VMEM((K, F2), jnp.float32) for _ in range(8)],
            pltpu.VMEM_SHARED((n_pad, F2), jnp.float32),
            pltpu.SemaphoreType.DMA,
            pltpu.SemaphoreType.DMA,
            *([pltpu.VMEM((64, F2), jnp.float32),
               pltpu.VMEM((64,), jnp.float32)]
              if mode == "diag" else []),
        ],
    )


def _make_densify(e_pad, nf_pad):
    SUP = 32
    nch_t = e_pad // K // NS
    nsup = nch_t // SUP
    rpt = nf_pad // NS
    mesh = plsc.VectorSubcoreMesh(core_axis_name="c", subcore_axis_name="s",
                                  num_cores=1)

    def body(rows_hbm, cols_hbm, vals_hbm, zeros_hbm, out_hbm,
             rw, cw, valw, idxw, acc, dsem):
        sid = lax.axis_index("s")
        r0 = pl.multiple_of(sid * rpt, 8)
        pltpu.sync_copy(zeros_hbm.at[pl.ds(r0, rpt)], acc.at[pl.ds(r0, rpt)])
        plsc.subcore_barrier()

        def sup(s8, _):
            base = pl.multiple_of(sid * nch_t + s8 * SUP, 8)
            pltpu.sync_copy(rows_hbm.at[pl.ds(base, SUP)], rw)
            pltpu.sync_copy(cols_hbm.at[pl.ds(base, SUP)], cw)
            pltpu.sync_copy(vals_hbm.at[pl.ds(base, SUP)], valw)
            for j in range(SUP):
                for g in range(K // LANES):
                    sl = pl.ds(g * LANES, LANES)
                    idxw[j, sl] = rw[j, sl] * F + cw[j, sl]
                pltpu.async_copy(valw.at[j], acc.at[idxw.at[j]], dsem,
                                 add=True)
            for j in range(SUP):
                pltpu.make_async_copy(zeros_hbm.at[pl.ds(0, K)], valw.at[j],
                                      dsem).wait()
            return 0

        lax.fori_loop(0, nsup, sup, 0)
        plsc.subcore_barrier()
        pltpu.sync_copy(acc.at[pl.ds(r0, rpt)], out_hbm.at[pl.ds(r0, rpt)])

    return pl.kernel(
        body,
        out_type=jax.ShapeDtypeStruct((nf_pad,), jnp.float32),
        mesh=mesh,
        compiler_params=pltpu.CompilerParams(use_tc_tiling_on_sc=False),
        scratch_types=[
            pltpu.VMEM((SUP, K), jnp.int32),
            pltpu.VMEM((SUP, K), jnp.int32),
            pltpu.VMEM((SUP, K), jnp.float32),
            pltpu.VMEM((SUP, K), jnp.int32),
            pltpu.VMEM_SHARED((nf_pad,), jnp.float32),
            pltpu.SemaphoreType.DMA,
        ],
    )


_BM = 1024


def _tc_mm1(z, w):
    np_ = z.shape[0]

    def body(z_ref, w_ref, o_ref):
        y = jnp.dot(z_ref[...], w_ref[...], preferred_element_type=jnp.float32)
        o_ref[0] = y[:, :F2]
        o_ref[1] = y[:, F2:]

    return pl.pallas_call(
        body,
        grid=(np_ // _BM,),
        in_specs=[
            pl.BlockSpec((_BM, F), lambda i: (i, 0)),
            pl.BlockSpec((F, F), lambda i: (0, 0)),
        ],
        out_specs=pl.BlockSpec((NC, _BM, F2), lambda i: (0, i, 0)),
        out_shape=jax.ShapeDtypeStruct((NC, np_, F2), jnp.float32),
    )(z, w)


def _tc_mm2(p, w):
    np_ = p.shape[1]

    def body(p_ref, w_ref, o_ref):
        x = jnp.concatenate([p_ref[0], p_ref[1]], axis=-1)
        x = jnp.maximum(x, 0.0)
        y = jnp.dot(x, w_ref[...], preferred_element_type=jnp.float32)
        o_ref[0] = y[:, :F2]
        o_ref[1] = y[:, F2:]

    return pl.pallas_call(
        body,
        grid=(np_ // _BM,),
        in_specs=[
            pl.BlockSpec((NC, _BM, F2), lambda i: (0, i, 0)),
            pl.BlockSpec((F, F), lambda i: (0, 0)),
        ],
        out_specs=pl.BlockSpec((NC, _BM, F2), lambda i: (0, i, 0)),
        out_shape=jax.ShapeDtypeStruct((NC, np_, F2), jnp.float32),
    )(p, w)


def _tc_scale(p, d):
    np_ = p.shape[1]

    def body(p_ref, d_ref, o_ref):
        o_ref[...] = p_ref[...] * d_ref[...][None]

    return pl.pallas_call(
        body,
        grid=(np_ // _BM,),
        in_specs=[
            pl.BlockSpec((NC, _BM, F2), lambda i: (0, i, 0)),
            pl.BlockSpec((_BM, 1), lambda i: (i, 0)),
        ],
        out_specs=pl.BlockSpec((NC, _BM, F2), lambda i: (0, i, 0)),
        out_shape=jax.ShapeDtypeStruct((NC, np_, F2), jnp.float32),
    )(p, d)


def _tc_final(p, n):
    bm = 2000

    def body(p_ref, o_ref):
        o_ref[...] = jnp.concatenate([p_ref[0], p_ref[1]], axis=-1)

    return pl.pallas_call(
        body,
        grid=(n // bm,),
        in_specs=[pl.BlockSpec((NC, bm, F2), lambda i: (0, i, 0))],
        out_specs=pl.BlockSpec((bm, F), lambda i: (i, 0)),
        out_shape=jax.ShapeDtypeStruct((n, F), jnp.float32),
    )(p)


def kernel(phi_indices, phi_values, phi_inverse_indices, phi_inverse_values,
           feature_indices, feature_values, W1, diag_w1, W2, diag_w2):
    n = diag_w1.shape[0]
    n_pad = _ceil_to(n, 512)
    e_pad = _ceil_to(phi_values.shape[0], NS * K * 8)
    ef_pad = _ceil_to(feature_values.shape[0], NS * K * 8)
    nf_pad = n_pad * F

    def pad_chunks(x, tot):
        return jnp.pad(x, (0, tot - x.shape[0])).reshape(tot // K, K)

    pr = pad_chunks(phi_indices[0], e_pad)
    pc = pad_chunks(phi_indices[1], e_pad)
    pv = pad_chunks(phi_values, e_pad)
    qr = pad_chunks(phi_inverse_indices[0], e_pad)
    qc = pad_chunks(phi_inverse_indices[1], e_pad)
    qv = pad_chunks(phi_inverse_values, e_pad)
    fr = pad_chunks(feature_indices[0], ef_pad)
    fc = pad_chunks(feature_indices[1], ef_pad)
    fv = pad_chunks(feature_values, ef_pad)

    zeros2d = jnp.zeros((n_pad, F2), jnp.float32)
    zeros1d = jnp.zeros((nf_pad,), jnp.float32)
    d1 = jnp.pad(diag_w1, (0, n_pad - n))
    d2 = jnp.pad(diag_w2, (0, n_pad - n))

    spmm_diag = _make_spmm(e_pad, n_pad, "diag")
    spmm_plain = _make_spmm(e_pad, n_pad, "plain")
    densify = _make_densify(ef_pad, nf_pad)

    z = densify(fr, fc, fv, zeros1d).reshape(n_pad, F)
    f1 = _tc_mm1(z, W1)
    p = spmm_diag(qr, qc, qv, f1, zeros2d, d1)
    p = spmm_plain(pr, pc, pv, p, zeros2d)
    f2 = _tc_mm2(p, W2)
    p = spmm_diag(qr, qc, qv, f2, zeros2d, d2)
    p = spmm_plain(pr, pc, pv, p, zeros2d)
    return _tc_final(p, n)
